# Initial kernel scaffold; baseline (speedup 1.0000x reference)
import jax
import jax.numpy as jnp
from jax.experimental import pallas as pl


def kernel(feat, edge_index, e_feat, edge_emb):
    raise NotImplementedError("write your pallas kernel here")


# speedup vs baseline: 4.8342x; 4.8342x over previous
import functools

import jax
import jax.numpy as jnp
from jax import lax
from jax.experimental import pallas as pl
from jax.experimental.pallas import tpu as pltpu
from jax.experimental.pallas import tpu_sc as plsc

N = 10000
E = 320000
HF = 128
NUM_ETYPES = 8
ALPHA = 0.05
ORDER = 5

NC = 2
NS = 16
NW = NC * NS
L = 16

N_PAD = 12288
RPT = N_PAD // NW
RPS = N_PAD // NS
CPT = 10240
B = 128
NB = CPT // B
EPT = E // NW
CR = 32

_mesh = plsc.VectorSubcoreMesh(core_axis_name="c", subcore_axis_name="s")
f32 = jnp.float32
i32 = jnp.int32


def _wid():
    return lax.axis_index("c") * NS + lax.axis_index("s")


def _zero_vmem_2d(ref, nrows):
    def row(r, _):
        for k in range(HF // L):
            ref[r, pl.ds(k * L, L)] = jnp.zeros((L,), f32)
        return 0
    lax.fori_loop(0, nrows, row, 0)


def _a1_body(ef_hbm, dst_hbm, emb_hbm, pe_out, sp_out,
             s_acc, pvm, tvm, dvm, pev, zb):
    cid = lax.axis_index("c")
    sid = lax.axis_index("s")
    wid = _wid()

    _zero_vmem_2d(zb, 1)
    for j in range(RPS // HF):
        pltpu.sync_copy(zb.at[0], s_acc.at[pl.ds(sid * RPS + j * HF, HF)])

    pvm[...] = jnp.zeros((L,), f32)
    pltpu.sync_copy(emb_hbm, pvm.at[pl.ds(0, NUM_ETYPES)])
    tv = pvm[...]
    lane = lax.iota(i32, L)
    pvm[...] = jnp.where(lane < NUM_ETYPES, jnp.exp(tv), 0.0)

    plsc.subcore_barrier()

    pltpu.sync_copy(ef_hbm.at[wid], tvm)
    pltpu.sync_copy(dst_hbm.at[wid], dvm)

    pvals = pvm[...]
    def blk_pe(b, _):
        for k in range(B // L):
            t = tvm[b, pl.ds(k * L, L)]
            acc = jnp.zeros((L,), f32)
            for tt in range(NUM_ETYPES):
                acc = jnp.where(t == tt, pvals[tt], acc)
            pev[b, pl.ds(k * L, L)] = acc
        return 0
    lax.fori_loop(0, NB, blk_pe, 0)

    pltpu.sync_copy(pev, pe_out.at[wid])

    def blk_sc(b, _):
        pltpu.sync_copy(pev.at[b], s_acc.at[dvm.at[b]], add=True)
        return 0
    lax.fori_loop(0, NB, blk_sc, 0)

    plsc.subcore_barrier()
    pltpu.sync_copy(s_acc.at[pl.ds(sid * RPS, RPS)],
                    sp_out.at[pl.ds(cid * N_PAD + sid * RPS, RPS)])


_a1 = pl.kernel(
    _a1_body,
    out_type=(
        jax.ShapeDtypeStruct((NW, NB, B), f32),
        jax.ShapeDtypeStruct((NC * N_PAD,), f32),
    ),
    mesh=_mesh,
    scratch_types=[
        pltpu.VMEM_SHARED((N_PAD,), f32),
        pltpu.VMEM((L,), f32),
        pltpu.VMEM((NB, B), i32),
        pltpu.VMEM((NB, B), i32),
        pltpu.VMEM((NB, B), f32),
        pltpu.VMEM((1, HF), f32),
    ],
)


def _a2_body(sp_hbm, feat_hbm, ginv_out, x0_out, s0_out,
             sv0, sv1, gv, av, fb):
    wid = _wid()
    base = wid * RPT

    pltpu.sync_copy(sp_hbm.at[pl.ds(base, RPT)], sv0)
    pltpu.sync_copy(sp_hbm.at[pl.ds(N_PAD + base, RPT)], sv1)

    def gvec(v, _):
        s = sv0[pl.ds(v * L, L)] + sv1[pl.ds(v * L, L)]
        gv[pl.ds(v * L, L)] = jnp.where(s > 0.0, 1.0 / s, 0.0)
        av[pl.ds(v * L, L)] = jnp.full((L,), ALPHA, f32)
        return 0
    lax.fori_loop(0, RPT // L, gvec, 0)
    pltpu.sync_copy(gv, ginv_out.at[pl.ds(base, RPT)])
    pltpu.sync_copy(av, s0_out.at[pl.ds(base, RPT)])

    def xblk(j, _):
        r0 = base + j * CR
        pltpu.sync_copy(feat_hbm.at[pl.ds(r0, CR)], fb)
        def row(r, _):
            for k in range(HF // L):
                fb[r, pl.ds(k * L, L)] = fb[r, pl.ds(k * L, L)] * ALPHA
            return 0
        lax.fori_loop(0, CR, row, 0)
        pltpu.sync_copy(fb, x0_out.at[pl.ds(r0, CR)])
        return 0
    lax.fori_loop(0, RPT // CR, xblk, 0)


_a2 = pl.kernel(
    _a2_body,
    out_type=(
        jax.ShapeDtypeStruct((N_PAD,), f32),
        jax.ShapeDtypeStruct((N_PAD, HF), f32),
        jax.ShapeDtypeStruct((N_PAD,), f32),
    ),
    mesh=_mesh,
    scratch_types=[
        pltpu.VMEM((RPT,), f32),
        pltpu.VMEM((RPT,), f32),
        pltpu.VMEM((RPT,), f32),
        pltpu.VMEM((RPT,), f32),
        pltpu.VMEM((CR, HF), f32),
    ],
)


def _b_body(x_hbm, s_hbm, src_hbm, dst_hbm, pe_hbm, ginv_hbm,
            xp_out, sp_out, a_out,
            accx, accs, rows, sval, srcv, dstv, pev, zb, *, with_a):
    cid = lax.axis_index("c")
    sid = lax.axis_index("s")
    wid = _wid()

    nz = zb.shape[0]
    _zero_vmem_2d(zb, nz)
    for i in range(RPS // nz):
        pltpu.sync_copy(zb, accx.at[pl.ds(sid * RPS + i * nz, nz)])
    for j in range(RPS // HF):
        pltpu.sync_copy(zb.at[0], accs.at[pl.ds(sid * RPS + j * HF, HF)])

    plsc.subcore_barrier()

    SB = srcv.shape[0]

    def hblk(h, _):
        pltpu.sync_copy(src_hbm.at[wid, pl.ds(h * SB, SB)], srcv)
        pltpu.sync_copy(dst_hbm.at[wid, pl.ds(h * SB, SB)], dstv)
        pltpu.sync_copy(pe_hbm.at[wid, pl.ds(h * SB, SB)], pev)

        def blk(b, _):
            pltpu.sync_copy(x_hbm.at[srcv.at[b]], rows)
            def egrp(q, _):
                pv = pev[b, pl.ds(q * L, L)]
                for i in range(L):
                    w = pv[i]
                    r = q * L + i
                    for k in range(HF // L):
                        rows[r, pl.ds(k * L, L)] = rows[r, pl.ds(k * L, L)] * w
                return 0
            lax.fori_loop(0, B // L, egrp, 0)
            pltpu.sync_copy(rows, accx.at[dstv.at[b]], add=True)

            pltpu.sync_copy(s_hbm.at[srcv.at[b]], sval)
            for k in range(B // L):
                sval[pl.ds(k * L, L)] = sval[pl.ds(k * L, L)] * pev[b, pl.ds(k * L, L)]
            pltpu.sync_copy(sval, accs.at[dstv.at[b]], add=True)

            if with_a:
                pltpu.sync_copy(ginv_hbm.at[dstv.at[b]], sval)
                for k in range(B // L):
                    sval[pl.ds(k * L, L)] = (sval[pl.ds(k * L, L)]
                                             * pev[b, pl.ds(k * L, L)])
                pltpu.sync_copy(sval, a_out.at[wid, h * SB + b])
            return 0
        lax.fori_loop(0, SB, blk, 0)
        return 0
    lax.fori_loop(0, NB // SB, hblk, 0)

    plsc.subcore_barrier()

    for i in range(RPS // nz):
        r0 = sid * RPS + i * nz
        pltpu.sync_copy(accx.at[pl.ds(r0, nz)],
                        xp_out.at[pl.ds(cid * N_PAD + r0, nz)])
    pltpu.sync_copy(accs.at[pl.ds(sid * RPS, RPS)],
                    sp_out.at[pl.ds(cid * N_PAD + sid * RPS, RPS)])


def _make_b(with_a):
    return pl.kernel(
        functools.partial(_b_body, with_a=with_a),
        out_type=(
            jax.ShapeDtypeStruct((NC * N_PAD, HF), f32),
            jax.ShapeDtypeStruct((NC * N_PAD,), f32),
            jax.ShapeDtypeStruct((NW, NB, B), f32),
        ),
        mesh=_mesh,
        scratch_types=[
            pltpu.VMEM_SHARED((N_PAD, HF), f32),
            pltpu.VMEM_SHARED((N_PAD,), f32),
            pltpu.VMEM((B, HF), f32),
            pltpu.VMEM((B,), f32),
            pltpu.VMEM((8, B), i32),
            pltpu.VMEM((8, B), i32),
            pltpu.VMEM((8, B), f32),
            pltpu.VMEM((16, HF), f32),
        ],
    )


_b_first = _make_b(True)
_b_rest = _make_b(False)


def _c_body(xp_hbm, sp_hbm, ginv_hbm, feat_hbm,
            x_out, s_out,
            sv0, sv1, gv, b0, b1, fb, *, final):
    wid = _wid()
    base = wid * RPT
    oma = 1.0 - ALPHA

    pltpu.sync_copy(sp_hbm.at[pl.ds(base, RPT)], sv0)
    pltpu.sync_copy(sp_hbm.at[pl.ds(N_PAD + base, RPT)], sv1)
    pltpu.sync_copy(ginv_hbm.at[pl.ds(base, RPT)], gv)

    def svec(v, _):
        s = (sv0[pl.ds(v * L, L)] + sv1[pl.ds(v * L, L)]) * gv[pl.ds(v * L, L)]
        sv0[pl.ds(v * L, L)] = s * oma + (ALPHA * ALPHA)
        return 0
    lax.fori_loop(0, RPT // L, svec, 0)
    pltpu.sync_copy(sv0, s_out.at[pl.ds(base, RPT)])
    if final:
        def rvec(v, _):
            sv1[pl.ds(v * L, L)] = 1.0 / sv0[pl.ds(v * L, L)]
            return 0
        lax.fori_loop(0, RPT // L, rvec, 0)

    def xblk(j, _):
        r0 = base + j * CR
        pltpu.sync_copy(xp_hbm.at[pl.ds(r0, CR)], b0)
        pltpu.sync_copy(xp_hbm.at[pl.ds(N_PAD + r0, CR)], b1)
        pltpu.sync_copy(feat_hbm.at[pl.ds(r0, CR)], fb)
        def rgrp(q, _):
            gvec = gv[pl.ds(j * CR + q * L, L)] * oma
            if final:
                rvecv = sv1[pl.ds(j * CR + q * L, L)]
            for i in range(L):
                g = gvec[i]
                r = q * L + i
                for k in range(HF // L):
                    v = ((b0[r, pl.ds(k * L, L)] + b1[r, pl.ds(k * L, L)]) * g
                         + fb[r, pl.ds(k * L, L)] * ALPHA)
                    if final:
                        v = v * rvecv[i]
                    b0[r, pl.ds(k * L, L)] = v
            return 0
        lax.fori_loop(0, CR // L, rgrp, 0)
        pltpu.sync_copy(b0, x_out.at[pl.ds(r0, CR)])
        return 0
    lax.fori_loop(0, RPT // CR, xblk, 0)


def _make_c(final):
    return pl.kernel(
        functools.partial(_c_body, final=final),
        out_type=(
            jax.ShapeDtypeStruct((N_PAD, HF), f32),
            jax.ShapeDtypeStruct((N_PAD,), f32),
        ),
        mesh=_mesh,
        scratch_types=[
            pltpu.VMEM((RPT,), f32),
            pltpu.VMEM((RPT,), f32),
            pltpu.VMEM((RPT,), f32),
            pltpu.VMEM((CR, HF), f32),
            pltpu.VMEM((CR, HF), f32),
            pltpu.VMEM((CR, HF), f32),
        ],
    )


_c_mid = _make_c(False)
_c_final = _make_c(True)


def kernel(feat, edge_index, e_feat, edge_emb):
    src = edge_index[0]
    dst = edge_index[1]

    src_c = jnp.pad(src.reshape(NW, EPT), ((0, 0), (0, CPT - EPT))
                    ).reshape(NW, NB, B)
    dst_c = jnp.pad(dst.reshape(NW, EPT), ((0, 0), (0, CPT - EPT))
                    ).reshape(NW, NB, B)
    ef_c = jnp.pad(e_feat.reshape(NW, EPT), ((0, 0), (0, CPT - EPT)),
                   constant_values=NUM_ETYPES).reshape(NW, NB, B)
    feat_pad = jnp.pad(feat, ((0, N_PAD - N), (0, 0)))
    emb_flat = edge_emb[:, 0]

    pe_c, sp = _a1(ef_c, dst_c, emb_flat)
    ginv, x, s = _a2(sp, feat_pad)

    a_c = None
    for i in range(ORDER):
        bk = _b_first if i == 0 else _b_rest
        xp, bsp, a_i = bk(x, s, src_c, dst_c, pe_c, ginv)
        if i == 0:
            a_c = a_i
        ck = _c_final if i == ORDER - 1 else _c_mid
        x, s = ck(xp, bsp, ginv, feat_pad)

    rst = x[:N]
    a = a_c.reshape(NW, CPT)[:, :EPT].reshape(E, 1)
    return (rst, a)


# speedup vs baseline: 6.1812x; 1.2787x over previous
import functools

import jax
import jax.numpy as jnp
from jax import lax
from jax.experimental import pallas as pl
from jax.experimental.pallas import tpu as pltpu
from jax.experimental.pallas import tpu_sc as plsc

N = 10000
E = 320000
HF = 128
NUM_ETYPES = 8
ALPHA = 0.05
ORDER = 5

NC = 2
NS = 16
NW = NC * NS
L = 16

N_PAD = 12288
RPT = N_PAD // NW
RPS = N_PAD // NS
CPT = 10240
B = 80
NB = CPT // B
SB = 16
EPT = E // NW
CR = 32
WB = 64

_mesh = plsc.VectorSubcoreMesh(core_axis_name="c", subcore_axis_name="s")
f32 = jnp.float32
i32 = jnp.int32


def _wid():
    return lax.axis_index("c") * NS + lax.axis_index("s")


def _zero_vmem_2d(ref, nrows):
    def row(r, _):
        for k in range(HF // L):
            ref[r, pl.ds(k * L, L)] = jnp.zeros((L,), f32)
        return 0
    lax.fori_loop(0, nrows, row, 0)


def _a1_body(ef_hbm, dst_hbm, emb_hbm, pe_out, sp_out,
             s_acc, pvm, tvm, dvm, pev, zb):
    cid = lax.axis_index("c")
    sid = lax.axis_index("s")
    wid = _wid()

    _zero_vmem_2d(zb, 1)
    for j in range(RPS // HF):
        pltpu.sync_copy(zb.at[0], s_acc.at[pl.ds(sid * RPS + j * HF, HF)])

    pvm[...] = jnp.zeros((L,), f32)
    pltpu.sync_copy(emb_hbm, pvm.at[pl.ds(0, NUM_ETYPES)])
    tv = pvm[...]
    lane = lax.iota(i32, L)
    pvm[...] = jnp.where(lane < NUM_ETYPES, jnp.exp(tv), 0.0)

    plsc.subcore_barrier()

    pltpu.sync_copy(ef_hbm.at[wid], tvm)
    pltpu.sync_copy(dst_hbm.at[wid], dvm)

    pvals = pvm[...]
    def blk_pe(b, _):
        for k in range(B // L):
            t = tvm[b, pl.ds(k * L, L)]
            acc = jnp.zeros((L,), f32)
            for tt in range(NUM_ETYPES):
                acc = jnp.where(t == tt, pvals[tt], acc)
            pev[b, pl.ds(k * L, L)] = acc
        return 0
    lax.fori_loop(0, NB, blk_pe, 0)

    pltpu.sync_copy(pev, pe_out.at[wid])

    def blk_sc(b, _):
        pltpu.sync_copy(pev.at[b], s_acc.at[dvm.at[b]], add=True)
        return 0
    lax.fori_loop(0, NB, blk_sc, 0)

    plsc.subcore_barrier()
    pltpu.sync_copy(s_acc.at[pl.ds(sid * RPS, RPS)],
                    sp_out.at[pl.ds(cid * N_PAD + sid * RPS, RPS)])


_a1 = pl.kernel(
    _a1_body,
    out_type=(
        jax.ShapeDtypeStruct((NW, NB, B), f32),
        jax.ShapeDtypeStruct((NC * N_PAD,), f32),
    ),
    mesh=_mesh,
    scratch_types=[
        pltpu.VMEM_SHARED((N_PAD,), f32),
        pltpu.VMEM((L,), f32),
        pltpu.VMEM((NB, B), i32),
        pltpu.VMEM((NB, B), i32),
        pltpu.VMEM((NB, B), f32),
        pltpu.VMEM((1, HF), f32),
    ],
)


def _a2_body(sp_hbm, feat_hbm, ginv_out, x0_out, s0_out,
             sv0, sv1, gv, av, fb):
    wid = _wid()
    base = wid * RPT

    pltpu.sync_copy(sp_hbm.at[pl.ds(base, RPT)], sv0)
    pltpu.sync_copy(sp_hbm.at[pl.ds(N_PAD + base, RPT)], sv1)

    def gvec(v, _):
        s = sv0[pl.ds(v * L, L)] + sv1[pl.ds(v * L, L)]
        gv[pl.ds(v * L, L)] = jnp.where(s > 0.0, 1.0 / s, 0.0)
        av[pl.ds(v * L, L)] = jnp.full((L,), ALPHA, f32)
        return 0
    lax.fori_loop(0, RPT // L, gvec, 0)
    pltpu.sync_copy(gv, ginv_out.at[pl.ds(base, RPT)])
    pltpu.sync_copy(av, s0_out.at[pl.ds(base, RPT)])

    def xblk(j, _):
        r0 = base + j * CR
        pltpu.sync_copy(feat_hbm.at[pl.ds(r0, CR)], fb)
        def row(r, _):
            for k in range(HF // L):
                fb[r, pl.ds(k * L, L)] = fb[r, pl.ds(k * L, L)] * ALPHA
            return 0
        lax.fori_loop(0, CR, row, 0)
        pltpu.sync_copy(fb, x0_out.at[pl.ds(r0, CR)])
        return 0
    lax.fori_loop(0, RPT // CR, xblk, 0)


_a2 = pl.kernel(
    _a2_body,
    out_type=(
        jax.ShapeDtypeStruct((N_PAD,), f32),
        jax.ShapeDtypeStruct((N_PAD, HF), f32),
        jax.ShapeDtypeStruct((N_PAD,), f32),
    ),
    mesh=_mesh,
    scratch_types=[
        pltpu.VMEM((RPT,), f32),
        pltpu.VMEM((RPT,), f32),
        pltpu.VMEM((RPT,), f32),
        pltpu.VMEM((RPT,), f32),
        pltpu.VMEM((CR, HF), f32),
    ],
)


def _b_body(x_hbm, s_hbm, src_hbm, dst_hbm, pe_hbm, ginv_hbm,
            xp_out, sp_out, a_out,
            accx, accs, rows, sval, aval, srcv, dstv, pev, zb,
            gsem, ssem, wsem, *, with_a):
    cid = lax.axis_index("c")
    sid = lax.axis_index("s")
    wid = _wid()

    nz = zb.shape[0]
    _zero_vmem_2d(zb, nz)
    descs = []
    for i in range(RPS // nz):
        descs.append(pltpu.async_copy(
            zb, accx.at[pl.ds(sid * RPS + i * nz, nz)], wsem))
    for j in range(RPS // HF):
        descs.append(pltpu.async_copy(
            zb.at[0], accs.at[pl.ds(sid * RPS + j * HF, HF)], wsem))
    for d in descs:
        d.wait()

    plsc.subcore_barrier()

    def hblk(h, _):
        pltpu.sync_copy(src_hbm.at[wid, pl.ds(h * SB, SB)], srcv)
        pltpu.sync_copy(dst_hbm.at[wid, pl.ds(h * SB, SB)], dstv)
        pltpu.sync_copy(pe_hbm.at[wid, pl.ds(h * SB, SB)], pev)

        for pb in range(2):
            pltpu.async_copy(x_hbm.at[srcv.at[pb]], rows.at[pb], gsem)
            pltpu.async_copy(s_hbm.at[srcv.at[pb]], sval.at[pb], ssem)

        def blk(b, _):
            par = b % 2
            pltpu.make_async_copy(x_hbm.at[srcv.at[b]], rows.at[par],
                                  gsem).wait()
            for q in range(B // L):
                pv = pev[b, pl.ds(q * L, L)]
                for i in range(L):
                    w = pv[i]
                    r = q * L + i
                    for k in range(HF // L):
                        rows[par, r, pl.ds(k * L, L)] = (
                            rows[par, r, pl.ds(k * L, L)] * w)
            pltpu.sync_copy(rows.at[par], accx.at[dstv.at[b]], add=True)

            pltpu.make_async_copy(s_hbm.at[srcv.at[b]], sval.at[par],
                                  ssem).wait()
            for q in range(B // L):
                sval[par, pl.ds(q * L, L)] = (sval[par, pl.ds(q * L, L)]
                                              * pev[b, pl.ds(q * L, L)])
            pltpu.sync_copy(sval.at[par], accs.at[dstv.at[b]], add=True)

            if with_a:
                pltpu.sync_copy(ginv_hbm.at[dstv.at[b]], aval)
                for q in range(B // L):
                    aval[pl.ds(q * L, L)] = (aval[pl.ds(q * L, L)]
                                             * pev[b, pl.ds(q * L, L)])
                pltpu.sync_copy(aval, a_out.at[wid, h * SB + b])

            @pl.when(b + 2 < SB)
            def _():
                pltpu.async_copy(x_hbm.at[srcv.at[b + 2]], rows.at[par], gsem)
                pltpu.async_copy(s_hbm.at[srcv.at[b + 2]], sval.at[par], ssem)
            return 0
        lax.fori_loop(0, SB, blk, 0)
        return 0
    lax.fori_loop(0, NB // SB, hblk, 0)

    plsc.subcore_barrier()

    descs = []
    for i in range(RPS // WB):
        r0 = sid * RPS + i * WB
        descs.append(pltpu.async_copy(
            accx.at[pl.ds(r0, WB)],
            xp_out.at[pl.ds(cid * N_PAD + r0, WB)], wsem))
    descs.append(pltpu.async_copy(
        accs.at[pl.ds(sid * RPS, RPS)],
        sp_out.at[pl.ds(cid * N_PAD + sid * RPS, RPS)], wsem))
    for d in descs:
        d.wait()


def _make_b(with_a):
    return pl.kernel(
        functools.partial(_b_body, with_a=with_a),
        out_type=(
            jax.ShapeDtypeStruct((NC * N_PAD, HF), f32),
            jax.ShapeDtypeStruct((NC * N_PAD,), f32),
            jax.ShapeDtypeStruct((NW, NB, B), f32),
        ),
        mesh=_mesh,
        scratch_types=[
            pltpu.VMEM_SHARED((N_PAD, HF), f32),
            pltpu.VMEM_SHARED((N_PAD,), f32),
            pltpu.VMEM((2, B, HF), f32),
            pltpu.VMEM((2, B), f32),
            pltpu.VMEM((B,), f32),
            pltpu.VMEM((SB, B), i32),
            pltpu.VMEM((SB, B), i32),
            pltpu.VMEM((SB, B), f32),
            pltpu.VMEM((16, HF), f32),
            pltpu.SemaphoreType.DMA,
            pltpu.SemaphoreType.DMA,
            pltpu.SemaphoreType.DMA,
        ],
    )


_b_first = _make_b(True)
_b_rest = _make_b(False)


def _c_body(xp_hbm, sp_hbm, ginv_hbm, feat_hbm,
            x_out, s_out,
            sv0, sv1, gv, b0, b1, fb, *, final):
    wid = _wid()
    base = wid * RPT
    oma = 1.0 - ALPHA

    pltpu.sync_copy(sp_hbm.at[pl.ds(base, RPT)], sv0)
    pltpu.sync_copy(sp_hbm.at[pl.ds(N_PAD + base, RPT)], sv1)
    pltpu.sync_copy(ginv_hbm.at[pl.ds(base, RPT)], gv)

    def svec(v, _):
        s = (sv0[pl.ds(v * L, L)] + sv1[pl.ds(v * L, L)]) * gv[pl.ds(v * L, L)]
        sv0[pl.ds(v * L, L)] = s * oma + (ALPHA * ALPHA)
        return 0
    lax.fori_loop(0, RPT // L, svec, 0)
    pltpu.sync_copy(sv0, s_out.at[pl.ds(base, RPT)])
    if final:
        def rvec(v, _):
            sv1[pl.ds(v * L, L)] = 1.0 / sv0[pl.ds(v * L, L)]
            return 0
        lax.fori_loop(0, RPT // L, rvec, 0)

    def xblk(j, _):
        r0 = base + j * CR
        pltpu.sync_copy(xp_hbm.at[pl.ds(r0, CR)], b0)
        pltpu.sync_copy(xp_hbm.at[pl.ds(N_PAD + r0, CR)], b1)
        pltpu.sync_copy(feat_hbm.at[pl.ds(r0, CR)], fb)
        def rgrp(q, _):
            gvec = gv[pl.ds(j * CR + q * L, L)] * oma
            if final:
                rvecv = sv1[pl.ds(j * CR + q * L, L)]
            for i in range(L):
                g = gvec[i]
                r = q * L + i
                for k in range(HF // L):
                    v = ((b0[r, pl.ds(k * L, L)] + b1[r, pl.ds(k * L, L)]) * g
                         + fb[r, pl.ds(k * L, L)] * ALPHA)
                    if final:
                        v = v * rvecv[i]
                    b0[r, pl.ds(k * L, L)] = v
            return 0
        lax.fori_loop(0, CR // L, rgrp, 0)
        pltpu.sync_copy(b0, x_out.at[pl.ds(r0, CR)])
        return 0
    lax.fori_loop(0, RPT // CR, xblk, 0)


def _make_c(final):
    return pl.kernel(
        functools.partial(_c_body, final=final),
        out_type=(
            jax.ShapeDtypeStruct((N_PAD, HF), f32),
            jax.ShapeDtypeStruct((N_PAD,), f32),
        ),
        mesh=_mesh,
        scratch_types=[
            pltpu.VMEM((RPT,), f32),
            pltpu.VMEM((RPT,), f32),
            pltpu.VMEM((RPT,), f32),
            pltpu.VMEM((CR, HF), f32),
            pltpu.VMEM((CR, HF), f32),
            pltpu.VMEM((CR, HF), f32),
        ],
    )


_c_mid = _make_c(False)
_c_final = _make_c(True)


def kernel(feat, edge_index, e_feat, edge_emb):
    src = edge_index[0]
    dst = edge_index[1]

    src_c = jnp.pad(src.reshape(NW, EPT), ((0, 0), (0, CPT - EPT))
                    ).reshape(NW, NB, B)
    dst_c = jnp.pad(dst.reshape(NW, EPT), ((0, 0), (0, CPT - EPT))
                    ).reshape(NW, NB, B)
    ef_c = jnp.pad(e_feat.reshape(NW, EPT), ((0, 0), (0, CPT - EPT)),
                   constant_values=NUM_ETYPES).reshape(NW, NB, B)
    feat_pad = jnp.pad(feat, ((0, N_PAD - N), (0, 0)))
    emb_flat = edge_emb[:, 0]

    pe_c, sp = _a1(ef_c, dst_c, emb_flat)
    ginv, x, s = _a2(sp, feat_pad)

    a_c = None
    for i in range(ORDER):
        bk = _b_first if i == 0 else _b_rest
        xp, bsp, a_i = bk(x, s, src_c, dst_c, pe_c, ginv)
        if i == 0:
            a_c = a_i
        ck = _c_final if i == ORDER - 1 else _c_mid
        x, s = ck(xp, bsp, ginv, feat_pad)

    rst = x[:N]
    a = a_c.reshape(NW, CPT)[:, :EPT].reshape(E, 1)
    return (rst, a)
